# baseline (speedup 1.0000x reference)
import jax
import jax.numpy as jnp
from jax.experimental import pallas as pl

BATCH = 1024
FEAT = 64
BITS = 16
ENC = FEAT * BITS
HID = 2048
CLASSES = 10
SEG_THRESH = 4.0

BM = 256


def _fused(xr_ref, w0_ref, w1_ref, w2_ref, ow_ref, out_ref):
    j = jax.lax.broadcasted_iota(jnp.int32, (1, ENC), 1)
    thr = (j % BITS).astype(jnp.float32) * (1.0 / (BITS - 1))
    a0 = (xr_ref[:] >= thr).astype(jnp.bfloat16)

    contract_rhs1 = (((1,), (1,)), ((), ()))

    z1 = jax.lax.dot_general(a0, w0_ref[:], contract_rhs1,
                             preferred_element_type=jnp.float32)
    a1 = (z1 >= SEG_THRESH).astype(jnp.bfloat16)
    z2 = jax.lax.dot_general(a1, w1_ref[:], contract_rhs1,
                             preferred_element_type=jnp.float32)
    a2 = (z2 >= SEG_THRESH).astype(jnp.bfloat16)
    z3 = jax.lax.dot_general(a2, w2_ref[:], contract_rhs1,
                             preferred_element_type=jnp.float32)
    a3 = (z3 >= SEG_THRESH).astype(jnp.bfloat16)

    s = jnp.dot(a1.astype(jnp.float32), ow_ref[0],
                preferred_element_type=jnp.float32)
    s = s + jnp.dot(a2.astype(jnp.float32), ow_ref[1],
                    preferred_element_type=jnp.float32)
    s = s + jnp.dot(a3.astype(jnp.float32), ow_ref[2],
                    preferred_element_type=jnp.float32)
    out_ref[:] = s


def kernel(x, W0, W1, W2, outW):
    xr = jnp.broadcast_to(x[:, :, None], (BATCH, FEAT, BITS)).reshape(BATCH, ENC)
    w0 = W0.astype(jnp.bfloat16)
    w1 = W1.astype(jnp.bfloat16)
    w2 = W2.astype(jnp.bfloat16)

    grid = (BATCH // BM,)
    return pl.pallas_call(
        _fused,
        grid=grid,
        in_specs=[
            pl.BlockSpec((BM, ENC), lambda i: (i, 0)),
            pl.BlockSpec((HID, ENC), lambda i: (0, 0)),
            pl.BlockSpec((HID, HID), lambda i: (0, 0)),
            pl.BlockSpec((HID, HID), lambda i: (0, 0)),
            pl.BlockSpec((3, HID, CLASSES), lambda i: (0, 0, 0)),
        ],
        out_specs=pl.BlockSpec((BM, CLASSES), lambda i: (i, 0)),
        out_shape=jax.ShapeDtypeStruct((BATCH, CLASSES), jnp.float32),
    )(xr, w0, w1, w2, outW)


# speedup vs baseline: 1.3146x; 1.3146x over previous
import jax
import jax.numpy as jnp
from jax.experimental import pallas as pl

BATCH = 1024
FEAT = 64
BITS = 16
ENC = FEAT * BITS
HID = 2048
CLASSES = 10
SEG_THRESH = 4.0

BM = 256


def _fused(xr_ref, w0_ref, w1_ref, w2_ref, ow_ref, out_ref):
    j = jax.lax.broadcasted_iota(jnp.int32, (1, ENC), 1)
    thr = (j % BITS).astype(jnp.float32) * (1.0 / (BITS - 1))
    a0 = (xr_ref[:] >= thr).astype(jnp.float32)

    contract_rhs1 = (((1,), (1,)), ((), ()))

    z1 = jax.lax.dot_general(a0, w0_ref[:], contract_rhs1,
                             preferred_element_type=jnp.float32)
    a1 = (z1 >= SEG_THRESH).astype(jnp.float32)
    z2 = jax.lax.dot_general(a1, w1_ref[:], contract_rhs1,
                             preferred_element_type=jnp.float32)
    a2 = (z2 >= SEG_THRESH).astype(jnp.float32)
    z3 = jax.lax.dot_general(a2, w2_ref[:], contract_rhs1,
                             preferred_element_type=jnp.float32)
    a3 = (z3 >= SEG_THRESH).astype(jnp.float32)

    s = jnp.dot(a1, ow_ref[0], preferred_element_type=jnp.float32)
    s = s + jnp.dot(a2, ow_ref[1], preferred_element_type=jnp.float32)
    s = s + jnp.dot(a3, ow_ref[2], preferred_element_type=jnp.float32)
    out_ref[:] = s


def kernel(x, W0, W1, W2, outW):
    xr = jnp.broadcast_to(x[:, :, None], (BATCH, FEAT, BITS)).reshape(BATCH, ENC)

    grid = (BATCH // BM,)
    return pl.pallas_call(
        _fused,
        grid=grid,
        in_specs=[
            pl.BlockSpec((BM, ENC), lambda i: (i, 0)),
            pl.BlockSpec((HID, ENC), lambda i: (0, 0)),
            pl.BlockSpec((HID, HID), lambda i: (0, 0)),
            pl.BlockSpec((HID, HID), lambda i: (0, 0)),
            pl.BlockSpec((3, HID, CLASSES), lambda i: (0, 0, 0)),
        ],
        out_specs=pl.BlockSpec((BM, CLASSES), lambda i: (i, 0)),
        out_shape=jax.ShapeDtypeStruct((BATCH, CLASSES), jnp.float32),
    )(xr, W0, W1, W2, outW)


# speedup vs baseline: 1.3966x; 1.0624x over previous
import jax
import jax.numpy as jnp
from jax.experimental import pallas as pl

BATCH = 1024
FEAT = 64
BITS = 16
ENC = FEAT * BITS
HID = 2048
CLASSES = 10
SEG_THRESH = 4.0

BM = 512


def _fused(x_ref, w0_ref, w1_ref, w2_ref, ow_ref, out_ref):
    k = jnp.floor(x_ref[:] * (BITS - 1.0))
    j = jax.lax.broadcasted_iota(jnp.int32, (FEAT, ENC), 1)
    f = jax.lax.broadcasted_iota(jnp.int32, (FEAT, ENC), 0)
    expand = (j // BITS == f).astype(jnp.float32)
    kr = jnp.dot(k, expand, preferred_element_type=jnp.float32)
    t = (jax.lax.broadcasted_iota(jnp.int32, (1, ENC), 1) % BITS).astype(
        jnp.float32)
    a0 = (kr >= t).astype(jnp.float32)

    contract_rhs1 = (((1,), (1,)), ((), ()))

    z1 = jax.lax.dot_general(a0, w0_ref[:], contract_rhs1,
                             preferred_element_type=jnp.float32)
    a1 = (z1 >= SEG_THRESH).astype(jnp.float32)
    z2 = jax.lax.dot_general(a1, w1_ref[:], contract_rhs1,
                             preferred_element_type=jnp.float32)
    a2 = (z2 >= SEG_THRESH).astype(jnp.float32)
    z3 = jax.lax.dot_general(a2, w2_ref[:], contract_rhs1,
                             preferred_element_type=jnp.float32)
    a3 = (z3 >= SEG_THRESH).astype(jnp.float32)

    s = jnp.dot(a1, ow_ref[0], preferred_element_type=jnp.float32)
    s = s + jnp.dot(a2, ow_ref[1], preferred_element_type=jnp.float32)
    s = s + jnp.dot(a3, ow_ref[2], preferred_element_type=jnp.float32)
    out_ref[:] = s


def kernel(x, W0, W1, W2, outW):
    grid = (BATCH // BM,)
    return pl.pallas_call(
        _fused,
        grid=grid,
        in_specs=[
            pl.BlockSpec((BM, FEAT), lambda i: (i, 0)),
            pl.BlockSpec((HID, ENC), lambda i: (0, 0)),
            pl.BlockSpec((HID, HID), lambda i: (0, 0)),
            pl.BlockSpec((HID, HID), lambda i: (0, 0)),
            pl.BlockSpec((3, HID, CLASSES), lambda i: (0, 0, 0)),
        ],
        out_specs=pl.BlockSpec((BM, CLASSES), lambda i: (i, 0)),
        out_shape=jax.ShapeDtypeStruct((BATCH, CLASSES), jnp.float32),
    )(x, W0, W1, W2, outW)


# speedup vs baseline: 1.5190x; 1.0876x over previous
import jax
import jax.numpy as jnp
from jax.experimental import pallas as pl
from jax.experimental.pallas import tpu as pltpu

BATCH = 1024
FEAT = 64
BITS = 16
ENC = FEAT * BITS
HID = 2048
CLASSES = 10
SEG_THRESH = 4.0

BN = 256
NT = HID // BN


def _fused(x_ref, w0_ref, w1_ref, w2_ref, ow_ref, out_ref,
           a0_ref, a1_ref, a2_ref):
    l = pl.program_id(0)
    j = pl.program_id(1)

    @pl.when(jnp.logical_and(l == 0, j == 0))
    def _init():
        k = jnp.floor(x_ref[:] * (BITS - 1.0))
        jf = jax.lax.broadcasted_iota(jnp.int32, (FEAT, ENC), 1)
        ff = jax.lax.broadcasted_iota(jnp.int32, (FEAT, ENC), 0)
        expand = (jf // BITS == ff).astype(jnp.float32)
        krT = jax.lax.dot_general(expand, k, (((0,), (1,)), ((), ())),
                                  preferred_element_type=jnp.float32)
        tT = (jax.lax.broadcasted_iota(jnp.int32, (ENC, 1), 0) % BITS
              ).astype(jnp.float32)
        a0_ref[:] = (krT >= tT).astype(jnp.bfloat16)
        out_ref[:] = jnp.zeros_like(out_ref)

    def _stage(w_ref, src_ref, dst_ref):
        zT = jax.lax.dot_general(w_ref[:], src_ref[:], (((1,), (0,)), ((), ())),
                                 preferred_element_type=jnp.float32)
        actT = (zT >= SEG_THRESH).astype(jnp.float32)
        if dst_ref is not None:
            dst_ref[pl.ds(j * BN, BN), :] = actT.astype(jnp.bfloat16)
        out_ref[:] += jax.lax.dot_general(
            actT, ow_ref[0], (((0,), (0,)), ((), ())),
            preferred_element_type=jnp.float32)

    @pl.when(l == 0)
    def _l0():
        _stage(w0_ref, a0_ref, a1_ref)

    @pl.when(l == 1)
    def _l1():
        _stage(w1_ref, a1_ref, a2_ref)

    @pl.when(l == 2)
    def _l2():
        _stage(w2_ref, a2_ref, None)


def kernel(x, W0, W1, W2, outW):
    grid = (3, NT)
    return pl.pallas_call(
        _fused,
        grid=grid,
        in_specs=[
            pl.BlockSpec((BATCH, FEAT), lambda l, j: (0, 0)),
            pl.BlockSpec((BN, ENC),
                         lambda l, j: (jnp.where(l == 0, j, NT - 1), 0)),
            pl.BlockSpec((BN, HID),
                         lambda l, j: (jnp.where(l < 1, 0,
                                                 jnp.where(l == 1, j, NT - 1)),
                                       0)),
            pl.BlockSpec((BN, HID),
                         lambda l, j: (jnp.where(l < 2, 0, j), 0)),
            pl.BlockSpec((1, BN, CLASSES), lambda l, j: (l, j, 0)),
        ],
        out_specs=pl.BlockSpec((BATCH, CLASSES), lambda l, j: (0, 0)),
        out_shape=jax.ShapeDtypeStruct((BATCH, CLASSES), jnp.float32),
        scratch_shapes=[
            pltpu.VMEM((ENC, BATCH), jnp.bfloat16),
            pltpu.VMEM((HID, BATCH), jnp.bfloat16),
            pltpu.VMEM((HID, BATCH), jnp.bfloat16),
        ],
    )(x, W0, W1, W2, outW)


# speedup vs baseline: 1.5259x; 1.0045x over previous
import jax
import jax.numpy as jnp
from jax.experimental import pallas as pl
from jax.experimental.pallas import tpu as pltpu

BATCH = 1024
FEAT = 64
BITS = 16
ENC = FEAT * BITS
HID = 2048
CLASSES = 10
SEG_THRESH = 4.0

BN = 256
NT = HID // BN


def _fused(x_ref, w0_ref, w1_ref, w2_ref, ow_ref, out_ref,
           a0_ref, a1_ref, a2_ref):
    l = pl.program_id(0)
    j = pl.program_id(1)

    @pl.when(jnp.logical_and(l == 0, j == 0))
    def _init():
        k = jnp.floor(x_ref[:] * (BITS - 1.0))
        jf = jax.lax.broadcasted_iota(jnp.int32, (FEAT, ENC), 1)
        ff = jax.lax.broadcasted_iota(jnp.int32, (FEAT, ENC), 0)
        expand = (jf // BITS == ff).astype(jnp.float32)
        krT = jax.lax.dot_general(expand, k, (((0,), (1,)), ((), ())),
                                  preferred_element_type=jnp.float32)
        tT = (jax.lax.broadcasted_iota(jnp.int32, (ENC, 1), 0) % BITS
              ).astype(jnp.float32)
        a0_ref[:] = (krT >= tT).astype(jnp.bfloat16)
        out_ref[:] = jnp.zeros_like(out_ref)

    def _stage(w_ref, src_ref, dst_ref):
        wb = w_ref[:].astype(jnp.bfloat16)
        zT = jax.lax.dot_general(wb, src_ref[:], (((1,), (0,)), ((), ())),
                                 preferred_element_type=jnp.float32)
        actT = (zT >= SEG_THRESH).astype(jnp.float32)
        if dst_ref is not None:
            dst_ref[pl.ds(j * BN, BN), :] = actT.astype(jnp.bfloat16)
        out_ref[:] += jax.lax.dot_general(
            actT, ow_ref[0], (((0,), (0,)), ((), ())),
            preferred_element_type=jnp.float32)

    @pl.when(l == 0)
    def _l0():
        _stage(w0_ref, a0_ref, a1_ref)

    @pl.when(l == 1)
    def _l1():
        _stage(w1_ref, a1_ref, a2_ref)

    @pl.when(l == 2)
    def _l2():
        _stage(w2_ref, a2_ref, None)


def kernel(x, W0, W1, W2, outW):
    grid = (3, NT)
    return pl.pallas_call(
        _fused,
        grid=grid,
        in_specs=[
            pl.BlockSpec((BATCH, FEAT), lambda l, j: (0, 0)),
            pl.BlockSpec((BN, ENC),
                         lambda l, j: (jnp.where(l == 0, j, NT - 1), 0)),
            pl.BlockSpec((BN, HID),
                         lambda l, j: (jnp.where(l < 1, 0,
                                                 jnp.where(l == 1, j, NT - 1)),
                                       0)),
            pl.BlockSpec((BN, HID),
                         lambda l, j: (jnp.where(l < 2, 0, j), 0)),
            pl.BlockSpec((1, BN, CLASSES), lambda l, j: (l, j, 0)),
        ],
        out_specs=pl.BlockSpec((BATCH, CLASSES), lambda l, j: (0, 0)),
        out_shape=jax.ShapeDtypeStruct((BATCH, CLASSES), jnp.float32),
        scratch_shapes=[
            pltpu.VMEM((ENC, BATCH), jnp.bfloat16),
            pltpu.VMEM((HID, BATCH), jnp.bfloat16),
            pltpu.VMEM((HID, BATCH), jnp.bfloat16),
        ],
    )(x, W0, W1, W2, outW)


# speedup vs baseline: 1.7111x; 1.1214x over previous
import jax
import jax.numpy as jnp
from jax.experimental import pallas as pl
from jax.experimental.pallas import tpu as pltpu

BATCH = 1024
FEAT = 64
BITS = 16
ENC = FEAT * BITS
HID = 2048
CLASSES = 10
SEG_THRESH = 4.0

BN = 512
NT = HID // BN


def _fused(x_ref, w0_ref, w1_ref, w2_ref, ow_ref, out_ref,
           a0_ref, a1_ref, a2_ref):
    l = pl.program_id(0)
    j = pl.program_id(1)

    @pl.when(jnp.logical_and(l == 0, j == 0))
    def _init():
        k = jnp.floor(x_ref[:] * (BITS - 1.0))
        jf = jax.lax.broadcasted_iota(jnp.int32, (FEAT, ENC), 1)
        ff = jax.lax.broadcasted_iota(jnp.int32, (FEAT, ENC), 0)
        expand = (jf // BITS == ff).astype(jnp.float32)
        krT = jax.lax.dot_general(expand, k, (((0,), (1,)), ((), ())),
                                  preferred_element_type=jnp.float32)
        tT = (jax.lax.broadcasted_iota(jnp.int32, (ENC, 1), 0) % BITS
              ).astype(jnp.float32)
        a0_ref[:] = (krT >= tT).astype(jnp.bfloat16)
        out_ref[:] = jnp.zeros_like(out_ref)

    def _stage(w_ref, src_ref, dst_ref):
        wb = w_ref[:].astype(jnp.bfloat16)
        zT = jax.lax.dot_general(wb, src_ref[:], (((1,), (0,)), ((), ())),
                                 preferred_element_type=jnp.float32)
        actT = (zT >= SEG_THRESH).astype(jnp.float32)
        if dst_ref is not None:
            dst_ref[pl.ds(j * BN, BN), :] = actT.astype(jnp.bfloat16)
        out_ref[:] += jax.lax.dot_general(
            actT, ow_ref[0], (((0,), (0,)), ((), ())),
            preferred_element_type=jnp.float32)

    @pl.when(l == 0)
    def _l0():
        _stage(w0_ref, a0_ref, a1_ref)

    @pl.when(l == 1)
    def _l1():
        _stage(w1_ref, a1_ref, a2_ref)

    @pl.when(l == 2)
    def _l2():
        _stage(w2_ref, a2_ref, None)


def kernel(x, W0, W1, W2, outW):
    grid = (3, NT)
    return pl.pallas_call(
        _fused,
        grid=grid,
        in_specs=[
            pl.BlockSpec((BATCH, FEAT), lambda l, j: (0, 0)),
            pl.BlockSpec((BN, ENC),
                         lambda l, j: (jnp.where(l == 0, j, NT - 1), 0)),
            pl.BlockSpec((BN, HID),
                         lambda l, j: (jnp.where(l < 1, 0,
                                                 jnp.where(l == 1, j, NT - 1)),
                                       0)),
            pl.BlockSpec((BN, HID),
                         lambda l, j: (jnp.where(l < 2, 0, j), 0)),
            pl.BlockSpec((1, BN, CLASSES), lambda l, j: (l, j, 0)),
        ],
        out_specs=pl.BlockSpec((BATCH, CLASSES), lambda l, j: (0, 0)),
        out_shape=jax.ShapeDtypeStruct((BATCH, CLASSES), jnp.float32),
        scratch_shapes=[
            pltpu.VMEM((ENC, BATCH), jnp.bfloat16),
            pltpu.VMEM((HID, BATCH), jnp.bfloat16),
            pltpu.VMEM((HID, BATCH), jnp.bfloat16),
        ],
    )(x, W0, W1, W2, outW)
